# baseline (speedup 1.0000x reference)
import jax
import jax.numpy as jnp
from jax import lax
from jax.experimental import pallas as pl
from jax.experimental.pallas import tpu as pltpu

_N = 10000
_ROWS = 8
_COLS = 1280

_ROTATIONS = ((13, 15, 26, 6), (17, 29, 16, 24))
_KEY_LO = 42


def _rotl(v, r):
    return lax.shift_left(v, jnp.uint32(r)) | lax.shift_right_logical(
        v, jnp.uint32(32 - r))


def _keep_mask(x1):
    k0 = jnp.uint32(0)
    k1 = jnp.uint32(_KEY_LO)
    ks = (k0, k1, k0 ^ k1 ^ jnp.uint32(0x1BD11BDA))
    x0 = jnp.zeros(x1.shape, jnp.uint32) + ks[0]
    x1 = x1 + ks[1]
    for i in range(5):
        for r in _ROTATIONS[i % 2]:
            x0 = x0 + x1
            x1 = _rotl(x1, r)
            x1 = x1 ^ x0
        x0 = x0 + ks[(i + 1) % 3]
        x1 = x1 + ks[(i + 2) % 3] + jnp.uint32(i + 1)
    bits = x0 ^ x1
    return lax.shift_right_logical(bits, jnp.uint32(9)) > jnp.uint32(419430)


def _body(x_ref, e_ref, y_ref, xo_ref, eo_ref, yo_ref, m1_ref, m2_ref):
    del x_ref, e_ref, y_ref, xo_ref, eo_ref, yo_ref
    cnt = (lax.broadcasted_iota(jnp.uint32, (_ROWS, _COLS), 0) * _COLS
           + lax.broadcasted_iota(jnp.uint32, (_ROWS, _COLS), 1))
    keep = _keep_mask(cnt)
    for r in range(_ROWS):
        row = jnp.reshape(keep[r:r + 1, :], (_COLS,))
        base = r * _COLS
        if base + _COLS <= _N:
            m1_ref[pl.ds(base, _COLS)] = row
            m2_ref[pl.ds(base, _COLS)] = row
        else:
            tail = _N - base
            part = lax.slice(row, (0,), (tail,))
            m1_ref[pl.ds(base, tail)] = part
            m2_ref[pl.ds(base, tail)] = part


def kernel(x, y, edge_index):
    x_out, e_out, y_out, m1, m2 = pl.pallas_call(
        _body,
        in_specs=[
            pl.BlockSpec(memory_space=pltpu.MemorySpace.HBM),
            pl.BlockSpec(memory_space=pltpu.MemorySpace.HBM),
            pl.BlockSpec(memory_space=pltpu.MemorySpace.HBM),
        ],
        out_specs=[
            pl.BlockSpec(memory_space=pltpu.MemorySpace.HBM),
            pl.BlockSpec(memory_space=pltpu.MemorySpace.HBM),
            pl.BlockSpec(memory_space=pltpu.MemorySpace.HBM),
            pl.BlockSpec(memory_space=pltpu.MemorySpace.VMEM),
            pl.BlockSpec(memory_space=pltpu.MemorySpace.VMEM),
        ],
        out_shape=[
            jax.ShapeDtypeStruct(x.shape, x.dtype),
            jax.ShapeDtypeStruct(edge_index.shape, edge_index.dtype),
            jax.ShapeDtypeStruct(y.shape, y.dtype),
            jax.ShapeDtypeStruct((_N,), jnp.bool_),
            jax.ShapeDtypeStruct((_N,), jnp.bool_),
        ],
        input_output_aliases={0: 0, 1: 1, 2: 2},
    )(x, edge_index, y)
    return (x_out, e_out, y_out, m1, m2)


# speedup vs baseline: 1.2667x; 1.2667x over previous
import jax
import jax.numpy as jnp
from jax import lax
from jax.experimental import pallas as pl
from jax.experimental.pallas import tpu as pltpu

_N = 10000
_ROWS = 8
_COLS = 1280

_ROTATIONS = ((13, 15, 26, 6), (17, 29, 16, 24))
_KEY_LO = 42


def _rotl(v, r):
    return lax.shift_left(v, jnp.uint32(r)) | lax.shift_right_logical(
        v, jnp.uint32(32 - r))


def _keep_mask(x1):
    k0 = jnp.uint32(0)
    k1 = jnp.uint32(_KEY_LO)
    ks = (k0, k1, k0 ^ k1 ^ jnp.uint32(0x1BD11BDA))
    x0 = jnp.zeros(x1.shape, jnp.uint32) + ks[0]
    x1 = x1 + ks[1]
    for i in range(5):
        for r in _ROTATIONS[i % 2]:
            x0 = x0 + x1
            x1 = _rotl(x1, r)
            x1 = x1 ^ x0
        x0 = x0 + ks[(i + 1) % 3]
        x1 = x1 + ks[(i + 2) % 3] + jnp.uint32(i + 1)
    bits = x0 ^ x1
    return lax.shift_right_logical(bits, jnp.uint32(9)) > jnp.uint32(419430)


def _body(x_ref, e_ref, y_ref, xo_ref, eo_ref, yo_ref, m1_ref, m2_ref,
          sem_x, sem_e, sem_y):
    cx = pltpu.make_async_copy(x_ref, xo_ref, sem_x)
    ce = pltpu.make_async_copy(e_ref, eo_ref, sem_e)
    cy = pltpu.make_async_copy(y_ref, yo_ref, sem_y)
    cx.start()
    ce.start()
    cy.start()
    cnt = (lax.broadcasted_iota(jnp.uint32, (_ROWS, _COLS), 0) * _COLS
           + lax.broadcasted_iota(jnp.uint32, (_ROWS, _COLS), 1))
    keep = _keep_mask(cnt)
    for r in range(_ROWS):
        row = jnp.reshape(keep[r:r + 1, :], (_COLS,))
        base = r * _COLS
        if base + _COLS <= _N:
            m1_ref[pl.ds(base, _COLS)] = row
            m2_ref[pl.ds(base, _COLS)] = row
        else:
            tail = _N - base
            part = lax.slice(row, (0,), (tail,))
            m1_ref[pl.ds(base, tail)] = part
            m2_ref[pl.ds(base, tail)] = part
    cx.wait()
    ce.wait()
    cy.wait()


def kernel(x, y, edge_index):
    x_out, e_out, y_out, m1, m2 = pl.pallas_call(
        _body,
        in_specs=[
            pl.BlockSpec(memory_space=pltpu.MemorySpace.VMEM),
            pl.BlockSpec(memory_space=pltpu.MemorySpace.VMEM),
            pl.BlockSpec(memory_space=pltpu.MemorySpace.VMEM),
        ],
        out_specs=[
            pl.BlockSpec(memory_space=pltpu.MemorySpace.HBM),
            pl.BlockSpec(memory_space=pltpu.MemorySpace.HBM),
            pl.BlockSpec(memory_space=pltpu.MemorySpace.HBM),
            pl.BlockSpec(memory_space=pltpu.MemorySpace.VMEM),
            pl.BlockSpec(memory_space=pltpu.MemorySpace.VMEM),
        ],
        out_shape=[
            jax.ShapeDtypeStruct(x.shape, x.dtype),
            jax.ShapeDtypeStruct(edge_index.shape, edge_index.dtype),
            jax.ShapeDtypeStruct(y.shape, y.dtype),
            jax.ShapeDtypeStruct((_N,), jnp.bool_),
            jax.ShapeDtypeStruct((_N,), jnp.bool_),
        ],
        scratch_shapes=[pltpu.SemaphoreType.DMA] * 3,
    )(x, edge_index, y)
    return (x_out, e_out, y_out, m1, m2)
